# baseline (speedup 1.0000x reference)
import functools

import jax
import jax.numpy as jnp
from jax import lax
from jax.experimental import pallas as pl
from jax.experimental.pallas import tpu as pltpu
from jax.experimental.pallas import tpu_sc as plsc

_B = 4096
_L = 200
_DIM = 64
_N = _B * _L
_NW = 32
_PER_W = _N // _NW
_CHUNK = 128
_GROUP = 4
_ROWS_G = _CHUNK * _GROUP
_NCHUNK = _PER_W // _CHUNK
_NGROUP = _PER_W // _ROWS_G

_mesh = plsc.VectorSubcoreMesh(core_axis_name="c", subcore_axis_name="s")


@functools.partial(
    pl.kernel,
    mesh=_mesh,
    out_type=jax.ShapeDtypeStruct((_N, _DIM), jnp.float32),
    scratch_types=[
        pltpu.VMEM((_NCHUNK, _CHUNK), jnp.int32),
        pltpu.VMEM((_ROWS_G, _DIM), jnp.float32),
        pltpu.VMEM((_ROWS_G, _DIM), jnp.float32),
        pltpu.SemaphoreType.DMA,
        pltpu.SemaphoreType.DMA,
    ],
    compiler_params=pltpu.CompilerParams(use_tc_tiling_on_sc=False),
)
def _gather_kernel(idx_hbm, table_hbm, out_hbm, idx_v, buf0, buf1, gsem, osem):
    wid = lax.axis_index("s") * 2 + lax.axis_index("c")
    base = wid * _PER_W

    pltpu.sync_copy(idx_hbm.at[wid], idx_v)

    def fire_gathers(g, buf):
        descs = []
        for j in range(_GROUP):
            cg = g * _GROUP + j
            descs.append(
                pltpu.async_copy(
                    table_hbm.at[idx_v.at[cg]],
                    buf.at[pl.ds(j * _CHUNK, _CHUNK)],
                    gsem,
                )
            )
        return descs

    def fire_out(g, buf):
        off = pl.multiple_of(base + g * _ROWS_G, _ROWS_G)
        return pltpu.async_copy(buf, out_hbm.at[pl.ds(off, _ROWS_G)], osem)

    def drain_out_pair():
        pltpu.make_async_copy(buf0, out_hbm.at[pl.ds(0, _ROWS_G)], osem).wait()
        pltpu.make_async_copy(buf1, out_hbm.at[pl.ds(0, _ROWS_G)], osem).wait()

    def body(g2, carry):
        g0 = g2 * 2
        g1 = g0 + 1

        @pl.when(g2 > 0)
        def _():
            drain_out_pair()

        d0 = fire_gathers(g0, buf0)
        d1 = fire_gathers(g1, buf1)
        for d in d0:
            d.wait()
        fire_out(g0, buf0)
        for d in d1:
            d.wait()
        fire_out(g1, buf1)
        return carry

    lax.fori_loop(0, _NGROUP // 2, body, 0)
    drain_out_pair()


def kernel(sets, E):
    flat = sets.reshape(_NW, _NCHUNK, _CHUNK)
    out = _gather_kernel(flat, E)
    return out.reshape(_B, _L, _DIM)
